# baseline (speedup 1.0000x reference)
import functools

import jax
import jax.numpy as jnp
from jax.experimental import pallas as pl
from jax.experimental.pallas import tpu as pltpu

N = 100000
E = 6400000


def _gat_jax(x, ei, W, a_s, a_d, b, heads, out_ch):
    n = x.shape[0]
    src, dst = ei[0], ei[1]
    h = (x @ W).reshape(n, heads, out_ch)
    as_ = jnp.sum(h * a_s[None, :, :], axis=-1)
    ad_ = jnp.sum(h * a_d[None, :, :], axis=-1)
    alpha = jax.nn.leaky_relu(as_[src] + ad_[dst], negative_slope=0.2)
    amax = jax.ops.segment_max(alpha, dst, num_segments=n)
    amax = jnp.where(jnp.isfinite(amax), amax, 0.0)
    ex = jnp.exp(alpha - amax[dst])
    den = jax.ops.segment_sum(ex, dst, num_segments=n)
    w = ex / (den[dst] + 1e-16)
    out = jax.ops.segment_sum(h[src] * w[:, :, None], dst, num_segments=n)
    return out.reshape(n, heads * out_ch) + b


def _bn_jax(x, g, b):
    m = jnp.mean(x, axis=0)
    v = jnp.var(x, axis=0)
    return (x - m) / jnp.sqrt(v + 1e-5) * g + b


def _drop_jax(x, key):
    keep = jax.random.bernoulli(key, 0.5, x.shape)
    return jnp.where(keep, x / 0.5, 0.0)


def _final_stage_body(actor_ref, pooled_ref, cw1_ref, cb1_ref, cw2_ref,
                      cb2_ref, cw3_ref, cb3_ref, prob_ref, value_ref,
                      logp_ref):
    t = jnp.tanh(actor_ref[...])
    m = jnp.max(t)
    e = jnp.exp(t - m)
    s = jnp.sum(e)
    prob_ref[...] = e / s
    logp_ref[...] = (t - m) - jnp.log(s)
    v = jnp.maximum(pooled_ref[...] @ cw1_ref[...] + cb1_ref[...], 0.0)
    v = jnp.maximum(v @ cw2_ref[...] + cb2_ref[...], 0.0)
    value_ref[...] = v @ cw3_ref[...] + cb3_ref[...]


def _final_stage(actor, pooled, p):
    out_shapes = (
        jax.ShapeDtypeStruct((1, N), jnp.float32),
        jax.ShapeDtypeStruct((1, 1), jnp.float32),
        jax.ShapeDtypeStruct((1, N), jnp.float32),
    )
    return pl.pallas_call(
        _final_stage_body,
        out_shape=out_shapes,
    )(actor.reshape(1, N), pooled, p['cw1'], p['cb1'].reshape(1, 32),
      p['cw2'], p['cb2'].reshape(1, 16), p['cw3'], p['cb3'].reshape(1, 1))


def kernel(x, edge_index, params):
    p = params
    h = _gat_jax(x, edge_index, p['W1'], p['as1'], p['ad1'], p['b1'], 4, 4)
    h = jax.nn.elu(_bn_jax(h, p['g1'], p['be1']))
    h = _drop_jax(h, jax.random.key(101))
    h = _gat_jax(h, edge_index, p['W2'], p['as2'], p['ad2'], p['b2'], 4, 16)
    h = jax.nn.elu(_bn_jax(h, p['g2'], p['be2']))
    h = _drop_jax(h, jax.random.key(202))
    pooled = jnp.mean(h, axis=0, keepdims=True)
    a = _gat_jax(h, edge_index, p['W3'], p['as3'], p['ad3'], p['b3'], 1, 32)
    a = _bn_jax(a, p['g3'], p['be3'])
    actor = _gat_jax(a, edge_index, p['W4'], p['as4'], p['ad4'], p['b4'], 1, 1)
    prob, value, log_prob = _final_stage(actor, pooled, p)
    return (prob, value, log_prob)


# speedup vs baseline: 162.4748x; 162.4748x over previous
import functools

import jax
import jax.numpy as jnp
from jax import lax
from jax.experimental import pallas as pl
from jax.experimental.pallas import tpu as pltpu
from jax.experimental.pallas import tpu_sc as plsc

N = 100000
E = 6400000
NPAD = 100352
TRASH = 100000
RPT = NPAD // 16
EPT = 200704
E_PAD = EPT * 32
ER = EPT // 128
SB = 256
NSB = EPT // SB
GRID = NPAD // 1024

_SC_PARAMS = pltpu.CompilerParams(
    needs_layout_passes=False, use_tc_tiling_on_sc=False)


def _make_edge_kernel(CU, K, TW):
    mesh = plsc.VectorSubcoreMesh(core_axis_name="c", subcore_axis_name="s",
                                  num_cores=2, num_subcores=16)
    u1d = CU == 1

    out_type = (
        jax.ShapeDtypeStruct((2, NPAD) if u1d else (2, NPAD, CU), jnp.float32),
        jax.ShapeDtypeStruct((2, K, NPAD), jnp.float32),
    )
    scratch = [
        pltpu.VMEM_SHARED((NPAD,) if u1d else (NPAD, CU), jnp.float32),
        *[pltpu.VMEM_SHARED((NPAD,), jnp.float32) for _ in range(K)],
        pltpu.VMEM((2, 2, 128), jnp.int32),
        pltpu.VMEM((4, 2, 128), jnp.int32),
        pltpu.VMEM((2, SB, TW), jnp.float32),
        *[pltpu.VMEM((2, SB), jnp.float32) for _ in range(K)],
        pltpu.VMEM((2, SB) if u1d else (2, SB, CU), jnp.float32),
        *[pltpu.VMEM((2, SB), jnp.float32) for _ in range(K)],
        pltpu.VMEM((K, 16), jnp.float32),
        pltpu.SemaphoreType.DMA,
        pltpu.SemaphoreType.DMA,
        pltpu.SemaphoreType.DMA,
        pltpu.SemaphoreType.DMA,
        pltpu.SemaphoreType.DMA,
        pltpu.SemaphoreType.DMA,
    ]

    @functools.partial(pl.kernel, out_type=out_type, mesh=mesh,
                       compiler_params=_SC_PARAMS, scratch_types=scratch)
    def ek(src_hbm, dst_hbm, tab_hbm, ad_hbm, gb_hbm, zu_hbm, zd_hbm,
           ou_hbm, od_hbm, *sc):
        acc_u, sc = sc[0], sc[1:]
        acc_d, sc = sc[:K], sc[K:]
        sidx, didx, rows, sc = sc[0], sc[1], sc[2], sc[3:]
        adv, sc = sc[:K], sc[K:]
        cu, sc = sc[0], sc[1:]
        cd, sc = sc[:K], sc[K:]
        gbv, sem_i0, sem_i1, sem_g0, sem_g1, sem_s0, sem_s1 = sc
        sem_i = (sem_i0, sem_i1)
        sem_g = (sem_g0, sem_g1)
        sem_s = (sem_s0, sem_s1)

        c = lax.axis_index("c")
        s = lax.axis_index("s")
        base = (s * 2 + c) * ER

        rpt_lo = s * RPT
        if u1d:
            pltpu.sync_copy(zd_hbm, acc_u.at[pl.ds(rpt_lo, RPT)])
        else:
            pltpu.sync_copy(zu_hbm, acc_u.at[pl.ds(rpt_lo, RPT), :])
        for k in range(K):
            pltpu.sync_copy(zd_hbm, acc_d[k].at[pl.ds(rpt_lo, RPT)])
        pltpu.sync_copy(gb_hbm, gbv)
        plsc.subcore_barrier()

        def idx_copies(x, s2, s4):
            r0 = base + x * 2
            return (
                (src_hbm.at[pl.ds(r0, 2), :], sidx.at[s2], sem_i[s2]),
                (dst_hbm.at[pl.ds(r0, 2), :], didx.at[s4], sem_i[s2]),
            )

        def gather_copies(s2, s4):
            out = []
            for j in range(2):
                out.append((tab_hbm.at[sidx.at[s2, j]],
                            rows.at[s2, pl.ds(j * 128, 128), :], sem_g[s2]))
                for k in range(K):
                    out.append((ad_hbm.at[k].at[didx.at[s4, j]],
                                adv[k].at[s2, pl.ds(j * 128, 128)], sem_g[s2]))
            return out

        def scatter_copies(s2, s4):
            out = []
            for j in range(2):
                csrc = (cu.at[s2, pl.ds(j * 128, 128)] if u1d
                        else cu.at[s2, pl.ds(j * 128, 128), :])
                cdst = (acc_u.at[didx.at[s4, j]])
                out.append((csrc, cdst, sem_s[s2]))
                for k in range(K):
                    out.append((cd[k].at[s2, pl.ds(j * 128, 128)],
                                acc_d[k].at[didx.at[s4, j]], sem_s[s2]))
            return out

        def fire(copies, add=False):
            for src, dst, sem in copies:
                pltpu.async_copy(src, dst, sem, add=add)

        def drain(copies):
            for src, dst, sem in copies:
                pltpu.make_async_copy(src, dst, sem).wait()

        colv = [jnp.full((16,), cc, jnp.int32) for cc in range(TW)]

        def compute(s2):
            rbuf = rows.at[s2]

            def grp(g, carry):
                e0 = g * 16
                e_ids = lax.iota(jnp.int32, 16) + e0
                svec = []
                for k in range(K):
                    a_s = plsc.load_gather(rbuf, [e_ids, colv[CU + k]])
                    a_d = adv[k][s2, pl.ds(e0, 16)]
                    al = a_s + a_d
                    al = jnp.where(al >= 0.0, al, al * 0.2)
                    sv = jnp.exp(al - gbv[k, :])
                    svec.append(sv)
                    cd[k][s2, pl.ds(e0, 16)] = sv
                if u1d:
                    hv = plsc.load_gather(rbuf, [e_ids, colv[0]])
                    cu[s2, pl.ds(e0, 16)] = hv * svec[0]
                else:
                    cbuf = cu.at[s2]
                    for cc in range(CU):
                        hv = plsc.load_gather(rbuf, [e_ids, colv[cc]])
                        plsc.store_scatter(cbuf, [e_ids, colv[cc]],
                                           hv * svec[cc * K // CU])
                return carry

            lax.fori_loop(0, SB // 16, grp, 0)

        fire(idx_copies(0, 0, 0))
        drain(idx_copies(0, 0, 0))
        fire(gather_copies(0, 0))
        fire(idx_copies(1, 1, 1))

        def iter4(it, carry):
            for u in range(4):
                s2, s4 = u % 2, u
                x = it * 4 + u

                @pl.when(x + 1 < NSB)
                def _():
                    drain(idx_copies(x + 1, (u + 1) % 2, (u + 1) % 4))
                    fire(gather_copies((u + 1) % 2, (u + 1) % 4))

                @pl.when(x >= 2)
                def _():
                    drain(scatter_copies(s2, (u + 2) % 4))

                drain(gather_copies(s2, s4))

                @pl.when(x + 2 < NSB)
                def _():
                    fire(idx_copies(x + 2, s2, (u + 2) % 4))

                compute(s2)
                fire(scatter_copies(s2, s4), add=True)
            return carry

        lax.fori_loop(0, NSB // 4, iter4, 0)
        drain(scatter_copies(0, 2))
        drain(scatter_copies(1, 3))
        plsc.subcore_barrier()
        if u1d:
            pltpu.sync_copy(acc_u.at[pl.ds(rpt_lo, RPT)],
                            ou_hbm.at[c, pl.ds(rpt_lo, RPT)])
        else:
            pltpu.sync_copy(acc_u.at[pl.ds(rpt_lo, RPT), :],
                            ou_hbm.at[c, pl.ds(rpt_lo, RPT), :])
        for k in range(K):
            pltpu.sync_copy(acc_d[k].at[pl.ds(rpt_lo, RPT)],
                            od_hbm.at[c, k, pl.ds(rpt_lo, RPT)])

    return ek


def _row_spec(w):
    return pl.BlockSpec((1024, w), lambda i: (i, 0))


def _full_spec(shape):
    nd = len(shape)
    return pl.BlockSpec(shape, lambda i, _n=nd: (0,) * _n)


def _u_spec(cu):
    if cu == 1:
        return pl.BlockSpec((2, 1024), lambda i: (0, i))
    return pl.BlockSpec((2, 1024, cu), lambda i: (0, i, 0))


def _d_spec(k):
    return pl.BlockSpec((2, k, 1024), lambda i: (0, 0, i))


def _grid_call(body, in_arrays, in_specs, out_widths):
    out_shape = tuple(jax.ShapeDtypeStruct((NPAD, w), jnp.float32)
                      for w in out_widths)
    out_specs = tuple(_row_spec(w) for w in out_widths)
    return pl.pallas_call(
        body, grid=(GRID,), in_specs=in_specs, out_specs=out_specs,
        out_shape=out_shape)(*in_arrays)


def _heads_as_ad(h, asw_ref, adw_ref, heads, ch):
    as_c, ad_c = [], []
    for hh in range(heads):
        hs = h[:, ch * hh:ch * hh + ch]
        as_c.append(jnp.sum(hs * asw_ref[hh:hh + 1, :], axis=1, keepdims=True))
        ad_c.append(jnp.sum(hs * adw_ref[hh:hh + 1, :], axis=1, keepdims=True))
    return jnp.concatenate(as_c, axis=1), jnp.concatenate(ad_c, axis=1)


def _zpad(m, w):
    return jnp.concatenate(
        [m, jnp.zeros((m.shape[0], w - m.shape[1]), jnp.float32)], axis=1)


def _prep1_body(x_ref, w_ref, asw_ref, adw_ref,
                ta_ref, tb_ref, aa_ref, ab_ref, as_ref, ad_ref):
    h = jnp.dot(x_ref[...], w_ref[...], preferred_element_type=jnp.float32)
    as_, ad_ = _heads_as_ad(h, asw_ref, adw_ref, 4, 4)
    ta_ref[...] = _zpad(jnp.concatenate([h[:, 0:8], as_[:, 0:2]], axis=1), 16)
    tb_ref[...] = _zpad(jnp.concatenate([h[:, 8:16], as_[:, 2:4]], axis=1), 16)
    aa_ref[...] = ad_[:, 0:2]
    ab_ref[...] = ad_[:, 2:4]
    as_ref[...] = as_
    ad_ref[...] = ad_


def _gb_body(as_ref, ad_ref, ms_ref, md_ref):
    i = pl.program_id(0)
    gr = i * 1024 + lax.broadcasted_iota(jnp.int32, (1024, 1), 0)
    big = jnp.float32(-3e38)
    pmax_s = jnp.max(jnp.where(gr < N, as_ref[...], big), axis=0,
                     keepdims=True)
    pmax_d = jnp.max(jnp.where(gr < N, ad_ref[...], big), axis=0,
                     keepdims=True)

    @pl.when(i == 0)
    def _():
        ms_ref[...] = pmax_s
        md_ref[...] = pmax_d

    @pl.when(i > 0)
    def _():
        ms_ref[...] = jnp.maximum(ms_ref[...], pmax_s)
        md_ref[...] = jnp.maximum(md_ref[...], pmax_d)


def _gb(as_full, ad_full, k):
    ms, md = pl.pallas_call(
        _gb_body, grid=(GRID,),
        in_specs=(_row_spec(k), _row_spec(k)),
        out_specs=(pl.BlockSpec((1, k), lambda i: (0, 0)),) * 2,
        out_shape=(jax.ShapeDtypeStruct((1, k), jnp.float32),) * 2,
    )(as_full, ad_full)
    return ms + md


def _comb1_body(ua_ref, da_ref, ub_ref, db_ref, b_ref, o_ref):
    ua, da = ua_ref[...], da_ref[...]
    ub, db = ub_ref[...], db_ref[...]
    sa, sb_ = ua[0] + ua[1], ub[0] + ub[1]
    da_s = da[0] + da[1]
    db_s = db[0] + db[1]
    o_ref[...] = jnp.concatenate(
        [sa[:, 0:4] / (da_s[0][:, None] + 1e-16),
         sa[:, 4:8] / (da_s[1][:, None] + 1e-16),
         sb_[:, 0:4] / (db_s[0][:, None] + 1e-16),
         sb_[:, 4:8] / (db_s[1][:, None] + 1e-16)],
        axis=1) + b_ref[...]


def _sums_body(x_ref, s1_ref, s2_ref):
    i = pl.program_id(0)
    gr = i * 1024 + lax.broadcasted_iota(jnp.int32, (1024, 1), 0)
    x = jnp.where(gr < N, x_ref[...], 0.0)
    ps1 = jnp.sum(x, axis=0, keepdims=True)
    ps2 = jnp.sum(x * x, axis=0, keepdims=True)

    @pl.when(i == 0)
    def _():
        s1_ref[...] = ps1
        s2_ref[...] = ps2

    @pl.when(i > 0)
    def _():
        s1_ref[...] += ps1
        s2_ref[...] += ps2


def _stats(h_full, w):
    return pl.pallas_call(
        _sums_body, grid=(GRID,),
        in_specs=(_row_spec(w),),
        out_specs=(pl.BlockSpec((1, w), lambda i: (0, 0)),) * 2,
        out_shape=(jax.ShapeDtypeStruct((1, w), jnp.float32),
                   jax.ShapeDtypeStruct((1, w), jnp.float32)),
    )(h_full)


def _mv(s1, s2):
    m = s1 * (1.0 / N)
    return m, s2 * (1.0 / N) - m * m


def _bn_elu_drop(h, s1, s2, g, be, mk):
    m, v = _mv(s1, s2)
    xb = (h - m) / jnp.sqrt(v + 1e-5) * g + be
    return jnp.where(xb > 0, xb, jnp.exp(xb) - 1.0) * mk


def _prep2_body(h_ref, m_ref, v_ref, g_ref, be_ref, mk_ref,
                w_ref, asw_ref, adw_ref,
                t0_ref, t1_ref, t2_ref, t3_ref,
                a0_ref, a1_ref, a2_ref, a3_ref, as_ref, ad_ref):
    t = _bn_elu_drop(h_ref[...], m_ref[...], v_ref[...], g_ref[...],
                     be_ref[...], mk_ref[...])
    h2 = jnp.dot(t, w_ref[...], preferred_element_type=jnp.float32)
    as_, ad_ = _heads_as_ad(h2, asw_ref, adw_ref, 4, 16)
    for hh, (tr, ar) in enumerate(zip((t0_ref, t1_ref, t2_ref, t3_ref),
                                      (a0_ref, a1_ref, a2_ref, a3_ref))):
        tr[...] = _zpad(jnp.concatenate(
            [h2[:, 16 * hh:16 * hh + 16], as_[:, hh:hh + 1]], axis=1), 24)
        ar[...] = ad_[:, hh:hh + 1]
    as_ref[...] = as_
    ad_ref[...] = ad_


def _comb2_body(u0_ref, d0_ref, u1_ref, d1_ref, u2_ref, d2_ref,
                u3_ref, d3_ref, b_ref, o_ref):
    cols = []
    for ur, dr in ((u0_ref, d0_ref), (u1_ref, d1_ref), (u2_ref, d2_ref),
                   (u3_ref, d3_ref)):
        uu, dd = ur[...], dr[...]
        us = uu[0] + uu[1]
        ds_ = dd[0] + dd[1]
        cols.append(us / (ds_[0][:, None] + 1e-16))
    o_ref[...] = jnp.concatenate(cols, axis=1) + b_ref[...]


def _prep3_body(h_ref, m_ref, v_ref, g_ref, be_ref, mk_ref,
                w_ref, asw_ref, adw_ref,
                td_ref, ta_ref, tb_ref, ad_t_ref, as_ref, ad_ref):
    t = _bn_elu_drop(h_ref[...], m_ref[...], v_ref[...], g_ref[...],
                     be_ref[...], mk_ref[...])
    h3 = jnp.dot(t, w_ref[...], preferred_element_type=jnp.float32)
    as_ = jnp.sum(h3 * asw_ref[...], axis=1, keepdims=True)
    ad_ = jnp.sum(h3 * adw_ref[...], axis=1, keepdims=True)
    td_ref[...] = t
    ta_ref[...] = _zpad(jnp.concatenate([h3[:, 0:16], as_], axis=1), 24)
    tb_ref[...] = _zpad(jnp.concatenate([h3[:, 16:32], as_], axis=1), 24)
    ad_t_ref[...] = ad_
    as_ref[...] = as_
    ad_ref[...] = ad_


def _comb3_body(ua_ref, da_ref, ub_ref, db_ref, b_ref, o_ref):
    ua, da = ua_ref[...], da_ref[...]
    ub, db = ub_ref[...], db_ref[...]
    sa = ua[0] + ua[1]
    sb_ = ub[0] + ub[1]
    da_s = da[0] + da[1]
    db_s = db[0] + db[1]
    o_ref[...] = jnp.concatenate(
        [sa / (da_s[0][:, None] + 1e-16), sb_ / (db_s[0][:, None] + 1e-16)],
        axis=1) + b_ref[...]


def _prep4_body(h_ref, m_ref, v_ref, g_ref, be_ref, w_ref, asw_ref, adw_ref,
                t4_ref, ad_t_ref, as_ref, ad_ref):
    m, v = _mv(m_ref[...], v_ref[...])
    xb = (h_ref[...] - m) / jnp.sqrt(v + 1e-5) * g_ref[...] + be_ref[...]
    h4 = jnp.dot(xb, w_ref[...], preferred_element_type=jnp.float32)
    as_ = h4 * asw_ref[...]
    ad_ = h4 * adw_ref[...]
    t4_ref[...] = _zpad(jnp.concatenate([h4, as_], axis=1), 8)
    ad_t_ref[...] = ad_
    as_ref[...] = as_
    ad_ref[...] = ad_


def _comb4_body(u_ref, d_ref, b_ref, o_ref):
    uu, dd = u_ref[...], d_ref[...]
    us = uu[0] + uu[1]
    ds_ = dd[0] + dd[1]
    o_ref[...] = (us / (ds_[0] + 1e-16) + b_ref[0, 0])[:, None]


def _final_body(actor_ref, psum_ref, cw1_ref, cb1_ref, cw2_ref,
                cb2_ref, cw3_ref, cb3_ref, prob_ref, value_ref, logp_ref):
    pooled_ref = psum_ref[...] * (1.0 / N)
    t = jnp.tanh(actor_ref[...])
    m = jnp.max(t)
    e = jnp.exp(t - m)
    sm = jnp.sum(e)
    prob_ref[...] = e / sm
    logp_ref[...] = (t - m) - jnp.log(sm)
    v = jnp.maximum(pooled_ref @ cw1_ref[...] + cb1_ref[...], 0.0)
    v = jnp.maximum(v @ cw2_ref[...] + cb2_ref[...], 0.0)
    value_ref[...] = v @ cw3_ref[...] + cb3_ref[...]


def kernel(x, edge_index, params):
    p = params
    f32 = jnp.float32

    xpad = jnp.pad(x, ((0, NPAD - N), (0, 0)))
    src = jnp.pad(edge_index[0].astype(jnp.int32), (0, E_PAD - E))
    dst = jnp.pad(edge_index[1].astype(jnp.int32), (0, E_PAD - E),
                  constant_values=TRASH)
    src2d = src.reshape(E_PAD // 128, 128)
    dst2d = dst.reshape(E_PAD // 128, 128)

    m1 = jnp.pad(jnp.where(jax.random.bernoulli(jax.random.key(101), 0.5,
                                                (N, 16)), 2.0, 0.0
                           ).astype(f32), ((0, NPAD - N), (0, 0)))
    m2 = jnp.pad(jnp.where(jax.random.bernoulli(jax.random.key(202), 0.5,
                                                (N, 64)), 2.0, 0.0
                           ).astype(f32), ((0, NPAD - N), (0, 0)))
    zu8 = jnp.zeros((RPT, 8), f32)
    zu16 = jnp.zeros((RPT, 16), f32)
    zd = jnp.zeros((RPT,), f32)

    ek_l1 = _make_edge_kernel(8, 2, 16)
    ek_l23 = _make_edge_kernel(16, 1, 24)
    ek_l4 = _make_edge_kernel(1, 1, 8)

    def gb_splat(gb_row, lo, k):
        return jnp.broadcast_to(gb_row[:, lo:lo + k].T, (k, 16)).astype(f32)

    t1a, t1b, ad1a, ad1b, as1, ad1 = _grid_call(
        _prep1_body,
        (xpad, p['W1'], p['as1'], p['ad1']),
        (_row_spec(4), _full_spec((4, 16)), _full_spec((4, 4)),
         _full_spec((4, 4))),
        (16, 16, 2, 2, 4, 4))
    gb1 = _gb(as1, ad1, 4)
    uA, dA = ek_l1(src2d, dst2d, t1a, ad1a.T, gb_splat(gb1, 0, 2), zu8, zd)
    uB, dB = ek_l1(src2d, dst2d, t1b, ad1b.T, gb_splat(gb1, 2, 2), zu8, zd)
    (h1,) = _grid_call(
        _comb1_body, (uA, dA, uB, dB, p['b1'].reshape(1, 16)),
        (_u_spec(8), _d_spec(2), _u_spec(8), _d_spec(2),
         _full_spec((1, 16))), (16,))
    m1s, v1s = _stats(h1, 16)

    t2 = _grid_call(
        _prep2_body,
        (h1, m1s, v1s, p['g1'].reshape(1, 16), p['be1'].reshape(1, 16), m1,
         p['W2'], p['as2'], p['ad2']),
        (_row_spec(16), _full_spec((1, 16)), _full_spec((1, 16)),
         _full_spec((1, 16)), _full_spec((1, 16)), _row_spec(16),
         _full_spec((16, 64)), _full_spec((4, 16)), _full_spec((4, 16))),
        (24, 24, 24, 24, 1, 1, 1, 1, 4, 4))
    t20, t21, t22, t23, a20, a21, a22, a23, as2, ad2 = t2
    gb2 = _gb(as2, ad2, 4)
    acc2 = [ek_l23(src2d, dst2d, tt, aa.T, gb_splat(gb2, hh, 1), zu16, zd)
            for hh, (tt, aa) in enumerate(zip((t20, t21, t22, t23),
                                              (a20, a21, a22, a23)))]
    (h2,) = _grid_call(
        _comb2_body,
        (acc2[0][0], acc2[0][1], acc2[1][0], acc2[1][1],
         acc2[2][0], acc2[2][1], acc2[3][0], acc2[3][1],
         p['b2'].reshape(1, 64)),
        (_u_spec(16), _d_spec(1)) * 4 + (_full_spec((1, 64)),), (64,))
    m2s, v2s = _stats(h2, 64)

    t2d, t3a, t3b, ad3t, as3, ad3 = _grid_call(
        _prep3_body,
        (h2, m2s, v2s, p['g2'].reshape(1, 64), p['be2'].reshape(1, 64), m2,
         p['W3'], p['as3'], p['ad3']),
        (_row_spec(64), _full_spec((1, 64)), _full_spec((1, 64)),
         _full_spec((1, 64)), _full_spec((1, 64)), _row_spec(64),
         _full_spec((64, 32)), _full_spec((1, 32)), _full_spec((1, 32))),
        (64, 24, 24, 1, 1, 1))
    psum, _psq = _stats(t2d, 64)
    gb3 = _gb(as3, ad3, 1)
    u3a, d3a = ek_l23(src2d, dst2d, t3a, ad3t.T, gb_splat(gb3, 0, 1),
                      zu16, zd)
    u3b, d3b = ek_l23(src2d, dst2d, t3b, ad3t.T, gb_splat(gb3, 0, 1),
                      zu16, zd)
    (a3,) = _grid_call(
        _comb3_body, (u3a, d3a, u3b, d3b, p['b3'].reshape(1, 32)),
        (_u_spec(16), _d_spec(1), _u_spec(16), _d_spec(1),
         _full_spec((1, 32))), (32,))
    m3s, v3s = _stats(a3, 32)

    t4, ad4t, as4, ad4 = _grid_call(
        _prep4_body,
        (a3, m3s, v3s, p['g3'].reshape(1, 32), p['be3'].reshape(1, 32),
         p['W4'], p['as4'], p['ad4']),
        (_row_spec(32), _full_spec((1, 32)), _full_spec((1, 32)),
         _full_spec((1, 32)), _full_spec((1, 32)), _full_spec((32, 1)),
         _full_spec((1, 1)), _full_spec((1, 1))),
        (8, 1, 1, 1))
    gb4 = _gb(as4, ad4, 1)
    u4, d4 = ek_l4(src2d, dst2d, t4, ad4t.T, gb_splat(gb4, 0, 1), zu16, zd)
    (actor,) = _grid_call(
        _comb4_body, (u4, d4, p['b4'].reshape(1, 1)),
        (_u_spec(1), _d_spec(1), _full_spec((1, 1))), (1,))

    prob, value, log_prob = pl.pallas_call(
        _final_body,
        out_shape=(jax.ShapeDtypeStruct((1, N), f32),
                   jax.ShapeDtypeStruct((1, 1), f32),
                   jax.ShapeDtypeStruct((1, N), f32)),
    )(actor[:N].reshape(1, N), psum, p['cw1'], p['cb1'].reshape(1, 32),
      p['cw2'], p['cb2'].reshape(1, 16), p['cw3'], p['cb3'].reshape(1, 1))
    return (prob, value, log_prob)


# speedup vs baseline: 221.9603x; 1.3661x over previous
import functools

import jax
import jax.numpy as jnp
from jax import lax
from jax.experimental import pallas as pl
from jax.experimental.pallas import tpu as pltpu
from jax.experimental.pallas import tpu_sc as plsc

N = 100000
E = 6400000
NPAD = 100352
TRASH = 100000
RPT = NPAD // 16
EPT = 200704
E_PAD = EPT * 32
ER = EPT // 128
SB = 256
NSB = EPT // SB
GRID = NPAD // 1024

_SC_PARAMS = pltpu.CompilerParams(
    needs_layout_passes=False, use_tc_tiling_on_sc=False)


def _make_edge_kernel(CU, K, TW):
    mesh = plsc.VectorSubcoreMesh(core_axis_name="c", subcore_axis_name="s",
                                  num_cores=2, num_subcores=16)
    u1d = CU == 1

    out_type = (
        jax.ShapeDtypeStruct((2, NPAD) if u1d else (2, NPAD, CU), jnp.float32),
        jax.ShapeDtypeStruct((2, K, NPAD), jnp.float32),
    )
    scratch = [
        pltpu.VMEM_SHARED((NPAD,) if u1d else (NPAD, CU), jnp.float32),
        *[pltpu.VMEM_SHARED((NPAD,), jnp.float32) for _ in range(K)],
        pltpu.VMEM((2, 2, 128), jnp.int32),
        pltpu.VMEM((4, 2, 128), jnp.int32),
        pltpu.VMEM((2, SB, TW), jnp.float32),
        *[pltpu.VMEM((2, SB), jnp.float32) for _ in range(K)],
        pltpu.VMEM((2, SB) if u1d else (2, SB, CU), jnp.float32),
        *[pltpu.VMEM((2, SB), jnp.float32) for _ in range(K)],
        pltpu.VMEM((K, 16), jnp.float32),
        pltpu.SemaphoreType.DMA,
        pltpu.SemaphoreType.DMA,
        pltpu.SemaphoreType.DMA,
        pltpu.SemaphoreType.DMA,
        pltpu.SemaphoreType.DMA,
        pltpu.SemaphoreType.DMA,
    ]

    @functools.partial(pl.kernel, out_type=out_type, mesh=mesh,
                       compiler_params=_SC_PARAMS, scratch_types=scratch)
    def ek(src_hbm, dst_hbm, tab_hbm, ad_hbm, gb_hbm, zu_hbm, zd_hbm,
           ou_hbm, od_hbm, *sc):
        acc_u, sc = sc[0], sc[1:]
        acc_d, sc = sc[:K], sc[K:]
        sidx, didx, rows, sc = sc[0], sc[1], sc[2], sc[3:]
        adv, sc = sc[:K], sc[K:]
        cu, sc = sc[0], sc[1:]
        cd, sc = sc[:K], sc[K:]
        gbv, sem_i0, sem_i1, sem_g0, sem_g1, sem_s0, sem_s1 = sc
        sem_i = (sem_i0, sem_i1)
        sem_g = (sem_g0, sem_g1)
        sem_s = (sem_s0, sem_s1)

        c = lax.axis_index("c")
        s = lax.axis_index("s")
        base = (s * 2 + c) * ER

        rpt_lo = s * RPT
        if u1d:
            pltpu.sync_copy(zd_hbm, acc_u.at[pl.ds(rpt_lo, RPT)])
        else:
            pltpu.sync_copy(zu_hbm, acc_u.at[pl.ds(rpt_lo, RPT), :])
        for k in range(K):
            pltpu.sync_copy(zd_hbm, acc_d[k].at[pl.ds(rpt_lo, RPT)])
        pltpu.sync_copy(gb_hbm, gbv)
        plsc.subcore_barrier()

        def idx_copies(x, s2, s4):
            r0 = base + x * 2
            return (
                (src_hbm.at[pl.ds(r0, 2), :], sidx.at[s2], sem_i[s2]),
                (dst_hbm.at[pl.ds(r0, 2), :], didx.at[s4], sem_i[s2]),
            )

        def gather_copies(s2, s4):
            out = []
            for j in range(2):
                out.append((tab_hbm.at[sidx.at[s2, j]],
                            rows.at[s2, pl.ds(j * 128, 128), :], sem_g[s2]))
                for k in range(K):
                    out.append((ad_hbm.at[k].at[didx.at[s4, j]],
                                adv[k].at[s2, pl.ds(j * 128, 128)], sem_g[s2]))
            return out

        def scatter_copies(s2, s4):
            out = []
            for j in range(2):
                csrc = (cu.at[s2, pl.ds(j * 128, 128)] if u1d
                        else cu.at[s2, pl.ds(j * 128, 128), :])
                cdst = (acc_u.at[didx.at[s4, j]])
                out.append((csrc, cdst, sem_s[s2]))
                for k in range(K):
                    out.append((cd[k].at[s2, pl.ds(j * 128, 128)],
                                acc_d[k].at[didx.at[s4, j]], sem_s[s2]))
            return out

        def fire(copies, add=False):
            for src, dst, sem in copies:
                pltpu.async_copy(src, dst, sem, add=add)

        def drain(copies):
            for src, dst, sem in copies:
                pltpu.make_async_copy(src, dst, sem).wait()

        colv = [jnp.full((16,), cc, jnp.int32) for cc in range(TW)]

        def compute(s2):
            rbuf = rows.at[s2]

            @plsc.parallel_loop(0, SB // 16, unroll=2)
            def grp(g):
                e0 = g * 16
                e_ids = lax.iota(jnp.int32, 16) + e0
                svec = []
                for k in range(K):
                    a_s = plsc.load_gather(rbuf, [e_ids, colv[CU + k]])
                    a_d = adv[k][s2, pl.ds(e0, 16)]
                    al = a_s + a_d
                    al = jnp.where(al >= 0.0, al, al * 0.2)
                    sv = jnp.exp(al - gbv[k, :])
                    svec.append(sv)
                    cd[k][s2, pl.ds(e0, 16)] = sv
                if u1d:
                    hv = plsc.load_gather(rbuf, [e_ids, colv[0]])
                    cu[s2, pl.ds(e0, 16)] = hv * svec[0]
                else:
                    cbuf = cu.at[s2]
                    for cc in range(CU):
                        hv = plsc.load_gather(rbuf, [e_ids, colv[cc]])
                        plsc.store_scatter(cbuf, [e_ids, colv[cc]],
                                           hv * svec[cc * K // CU])

        fire(idx_copies(0, 0, 0))
        drain(idx_copies(0, 0, 0))
        fire(gather_copies(0, 0))
        fire(idx_copies(1, 1, 1))

        def iter4(it, carry):
            for u in range(4):
                s2, s4 = u % 2, u
                x = it * 4 + u

                @pl.when(x + 1 < NSB)
                def _():
                    drain(idx_copies(x + 1, (u + 1) % 2, (u + 1) % 4))
                    fire(gather_copies((u + 1) % 2, (u + 1) % 4))

                @pl.when(x >= 2)
                def _():
                    drain(scatter_copies(s2, (u + 2) % 4))

                drain(gather_copies(s2, s4))

                @pl.when(x + 2 < NSB)
                def _():
                    fire(idx_copies(x + 2, s2, (u + 2) % 4))

                compute(s2)
                fire(scatter_copies(s2, s4), add=True)
            return carry

        lax.fori_loop(0, NSB // 4, iter4, 0)
        drain(scatter_copies(0, 2))
        drain(scatter_copies(1, 3))
        plsc.subcore_barrier()
        if u1d:
            pltpu.sync_copy(acc_u.at[pl.ds(rpt_lo, RPT)],
                            ou_hbm.at[c, pl.ds(rpt_lo, RPT)])
        else:
            pltpu.sync_copy(acc_u.at[pl.ds(rpt_lo, RPT), :],
                            ou_hbm.at[c, pl.ds(rpt_lo, RPT), :])
        for k in range(K):
            pltpu.sync_copy(acc_d[k].at[pl.ds(rpt_lo, RPT)],
                            od_hbm.at[c, k, pl.ds(rpt_lo, RPT)])

    return ek


def _row_spec(w):
    return pl.BlockSpec((1024, w), lambda i: (i, 0))


def _full_spec(shape):
    nd = len(shape)
    return pl.BlockSpec(shape, lambda i, _n=nd: (0,) * _n)


def _u_spec(cu):
    if cu == 1:
        return pl.BlockSpec((2, 1024), lambda i: (0, i))
    return pl.BlockSpec((2, 1024, cu), lambda i: (0, i, 0))


def _d_spec(k):
    return pl.BlockSpec((2, k, 1024), lambda i: (0, 0, i))


def _grid_call(body, in_arrays, in_specs, out_widths):
    out_shape = tuple(jax.ShapeDtypeStruct((NPAD, w), jnp.float32)
                      for w in out_widths)
    out_specs = tuple(_row_spec(w) for w in out_widths)
    return pl.pallas_call(
        body, grid=(GRID,), in_specs=in_specs, out_specs=out_specs,
        out_shape=out_shape)(*in_arrays)


def _heads_as_ad(h, asw_ref, adw_ref, heads, ch):
    as_c, ad_c = [], []
    for hh in range(heads):
        hs = h[:, ch * hh:ch * hh + ch]
        as_c.append(jnp.sum(hs * asw_ref[hh:hh + 1, :], axis=1, keepdims=True))
        ad_c.append(jnp.sum(hs * adw_ref[hh:hh + 1, :], axis=1, keepdims=True))
    return jnp.concatenate(as_c, axis=1), jnp.concatenate(ad_c, axis=1)


def _zpad(m, w):
    return jnp.concatenate(
        [m, jnp.zeros((m.shape[0], w - m.shape[1]), jnp.float32)], axis=1)


def _prep1_body(x_ref, w_ref, asw_ref, adw_ref,
                ta_ref, tb_ref, aa_ref, ab_ref, as_ref, ad_ref):
    h = jnp.dot(x_ref[...], w_ref[...], preferred_element_type=jnp.float32)
    as_, ad_ = _heads_as_ad(h, asw_ref, adw_ref, 4, 4)
    ta_ref[...] = _zpad(jnp.concatenate([h[:, 0:8], as_[:, 0:2]], axis=1), 16)
    tb_ref[...] = _zpad(jnp.concatenate([h[:, 8:16], as_[:, 2:4]], axis=1), 16)
    aa_ref[...] = ad_[:, 0:2]
    ab_ref[...] = ad_[:, 2:4]
    as_ref[...] = as_
    ad_ref[...] = ad_


def _gb_body(as_ref, ad_ref, ms_ref, md_ref):
    i = pl.program_id(0)
    gr = i * 1024 + lax.broadcasted_iota(jnp.int32, (1024, 1), 0)
    big = jnp.float32(-3e38)
    pmax_s = jnp.max(jnp.where(gr < N, as_ref[...], big), axis=0,
                     keepdims=True)
    pmax_d = jnp.max(jnp.where(gr < N, ad_ref[...], big), axis=0,
                     keepdims=True)

    @pl.when(i == 0)
    def _():
        ms_ref[...] = pmax_s
        md_ref[...] = pmax_d

    @pl.when(i > 0)
    def _():
        ms_ref[...] = jnp.maximum(ms_ref[...], pmax_s)
        md_ref[...] = jnp.maximum(md_ref[...], pmax_d)


def _gb(as_full, ad_full, k):
    ms, md = pl.pallas_call(
        _gb_body, grid=(GRID,),
        in_specs=(_row_spec(k), _row_spec(k)),
        out_specs=(pl.BlockSpec((1, k), lambda i: (0, 0)),) * 2,
        out_shape=(jax.ShapeDtypeStruct((1, k), jnp.float32),) * 2,
    )(as_full, ad_full)
    return ms + md


def _comb1_body(ua_ref, da_ref, ub_ref, db_ref, b_ref, o_ref):
    ua, da = ua_ref[...], da_ref[...]
    ub, db = ub_ref[...], db_ref[...]
    sa, sb_ = ua[0] + ua[1], ub[0] + ub[1]
    da_s = da[0] + da[1]
    db_s = db[0] + db[1]
    o_ref[...] = jnp.concatenate(
        [sa[:, 0:4] / (da_s[0][:, None] + 1e-16),
         sa[:, 4:8] / (da_s[1][:, None] + 1e-16),
         sb_[:, 0:4] / (db_s[0][:, None] + 1e-16),
         sb_[:, 4:8] / (db_s[1][:, None] + 1e-16)],
        axis=1) + b_ref[...]


def _sums_body(x_ref, s1_ref, s2_ref):
    i = pl.program_id(0)
    gr = i * 1024 + lax.broadcasted_iota(jnp.int32, (1024, 1), 0)
    x = jnp.where(gr < N, x_ref[...], 0.0)
    ps1 = jnp.sum(x, axis=0, keepdims=True)
    ps2 = jnp.sum(x * x, axis=0, keepdims=True)

    @pl.when(i == 0)
    def _():
        s1_ref[...] = ps1
        s2_ref[...] = ps2

    @pl.when(i > 0)
    def _():
        s1_ref[...] += ps1
        s2_ref[...] += ps2


def _stats(h_full, w):
    return pl.pallas_call(
        _sums_body, grid=(GRID,),
        in_specs=(_row_spec(w),),
        out_specs=(pl.BlockSpec((1, w), lambda i: (0, 0)),) * 2,
        out_shape=(jax.ShapeDtypeStruct((1, w), jnp.float32),
                   jax.ShapeDtypeStruct((1, w), jnp.float32)),
    )(h_full)


def _mv(s1, s2):
    m = s1 * (1.0 / N)
    return m, s2 * (1.0 / N) - m * m


def _bn_elu_drop(h, s1, s2, g, be, mk):
    m, v = _mv(s1, s2)
    xb = (h - m) / jnp.sqrt(v + 1e-5) * g + be
    return jnp.where(xb > 0, xb, jnp.exp(xb) - 1.0) * mk


def _prep2_body(h_ref, m_ref, v_ref, g_ref, be_ref, mk_ref,
                w_ref, asw_ref, adw_ref,
                t0_ref, t1_ref, t2_ref, t3_ref,
                a0_ref, a1_ref, a2_ref, a3_ref, as_ref, ad_ref):
    t = _bn_elu_drop(h_ref[...], m_ref[...], v_ref[...], g_ref[...],
                     be_ref[...], mk_ref[...])
    h2 = jnp.dot(t, w_ref[...], preferred_element_type=jnp.float32)
    as_, ad_ = _heads_as_ad(h2, asw_ref, adw_ref, 4, 16)
    for hh, (tr, ar) in enumerate(zip((t0_ref, t1_ref, t2_ref, t3_ref),
                                      (a0_ref, a1_ref, a2_ref, a3_ref))):
        tr[...] = _zpad(jnp.concatenate(
            [h2[:, 16 * hh:16 * hh + 16], as_[:, hh:hh + 1]], axis=1), 24)
        ar[...] = ad_[:, hh:hh + 1]
    as_ref[...] = as_
    ad_ref[...] = ad_


def _comb2_body(u0_ref, d0_ref, u1_ref, d1_ref, u2_ref, d2_ref,
                u3_ref, d3_ref, b_ref, o_ref):
    cols = []
    for ur, dr in ((u0_ref, d0_ref), (u1_ref, d1_ref), (u2_ref, d2_ref),
                   (u3_ref, d3_ref)):
        uu, dd = ur[...], dr[...]
        us = uu[0] + uu[1]
        ds_ = dd[0] + dd[1]
        cols.append(us / (ds_[0][:, None] + 1e-16))
    o_ref[...] = jnp.concatenate(cols, axis=1) + b_ref[...]


def _prep3_body(h_ref, m_ref, v_ref, g_ref, be_ref, mk_ref,
                w_ref, asw_ref, adw_ref,
                td_ref, ta_ref, tb_ref, ad_t_ref, as_ref, ad_ref):
    t = _bn_elu_drop(h_ref[...], m_ref[...], v_ref[...], g_ref[...],
                     be_ref[...], mk_ref[...])
    h3 = jnp.dot(t, w_ref[...], preferred_element_type=jnp.float32)
    as_ = jnp.sum(h3 * asw_ref[...], axis=1, keepdims=True)
    ad_ = jnp.sum(h3 * adw_ref[...], axis=1, keepdims=True)
    td_ref[...] = t
    ta_ref[...] = _zpad(jnp.concatenate([h3[:, 0:16], as_], axis=1), 24)
    tb_ref[...] = _zpad(jnp.concatenate([h3[:, 16:32], as_], axis=1), 24)
    ad_t_ref[...] = ad_
    as_ref[...] = as_
    ad_ref[...] = ad_


def _comb3_body(ua_ref, da_ref, ub_ref, db_ref, b_ref, o_ref):
    ua, da = ua_ref[...], da_ref[...]
    ub, db = ub_ref[...], db_ref[...]
    sa = ua[0] + ua[1]
    sb_ = ub[0] + ub[1]
    da_s = da[0] + da[1]
    db_s = db[0] + db[1]
    o_ref[...] = jnp.concatenate(
        [sa / (da_s[0][:, None] + 1e-16), sb_ / (db_s[0][:, None] + 1e-16)],
        axis=1) + b_ref[...]


def _prep4_body(h_ref, m_ref, v_ref, g_ref, be_ref, w_ref, asw_ref, adw_ref,
                t4_ref, ad_t_ref, as_ref, ad_ref):
    m, v = _mv(m_ref[...], v_ref[...])
    xb = (h_ref[...] - m) / jnp.sqrt(v + 1e-5) * g_ref[...] + be_ref[...]
    h4 = jnp.dot(xb, w_ref[...], preferred_element_type=jnp.float32)
    as_ = h4 * asw_ref[...]
    ad_ = h4 * adw_ref[...]
    t4_ref[...] = _zpad(jnp.concatenate([h4, as_], axis=1), 8)
    ad_t_ref[...] = ad_
    as_ref[...] = as_
    ad_ref[...] = ad_


def _comb4_body(u_ref, d_ref, b_ref, o_ref):
    uu, dd = u_ref[...], d_ref[...]
    us = uu[0] + uu[1]
    ds_ = dd[0] + dd[1]
    o_ref[...] = (us / (ds_[0] + 1e-16) + b_ref[0, 0])[:, None]


def _final_body(actor_ref, psum_ref, cw1_ref, cb1_ref, cw2_ref,
                cb2_ref, cw3_ref, cb3_ref, prob_ref, value_ref, logp_ref):
    pooled_ref = psum_ref[...] * (1.0 / N)
    t = jnp.tanh(actor_ref[...])
    m = jnp.max(t)
    e = jnp.exp(t - m)
    sm = jnp.sum(e)
    prob_ref[...] = e / sm
    logp_ref[...] = (t - m) - jnp.log(sm)
    v = jnp.maximum(pooled_ref @ cw1_ref[...] + cb1_ref[...], 0.0)
    v = jnp.maximum(v @ cw2_ref[...] + cb2_ref[...], 0.0)
    value_ref[...] = v @ cw3_ref[...] + cb3_ref[...]


def kernel(x, edge_index, params):
    p = params
    f32 = jnp.float32

    xpad = jnp.pad(x, ((0, NPAD - N), (0, 0)))
    src = jnp.pad(edge_index[0].astype(jnp.int32), (0, E_PAD - E))
    dst = jnp.pad(edge_index[1].astype(jnp.int32), (0, E_PAD - E),
                  constant_values=TRASH)
    src2d = src.reshape(E_PAD // 128, 128)
    dst2d = dst.reshape(E_PAD // 128, 128)

    m1 = jnp.pad(jnp.where(jax.random.bernoulli(jax.random.key(101), 0.5,
                                                (N, 16)), 2.0, 0.0
                           ).astype(f32), ((0, NPAD - N), (0, 0)))
    m2 = jnp.pad(jnp.where(jax.random.bernoulli(jax.random.key(202), 0.5,
                                                (N, 64)), 2.0, 0.0
                           ).astype(f32), ((0, NPAD - N), (0, 0)))
    zu8 = jnp.zeros((RPT, 8), f32)
    zu16 = jnp.zeros((RPT, 16), f32)
    zd = jnp.zeros((RPT,), f32)

    ek_l1 = _make_edge_kernel(8, 2, 16)
    ek_l23 = _make_edge_kernel(16, 1, 24)
    ek_l4 = _make_edge_kernel(1, 1, 8)

    def gb_splat(gb_row, lo, k):
        return jnp.broadcast_to(gb_row[:, lo:lo + k].T, (k, 16)).astype(f32)

    t1a, t1b, ad1a, ad1b, as1, ad1 = _grid_call(
        _prep1_body,
        (xpad, p['W1'], p['as1'], p['ad1']),
        (_row_spec(4), _full_spec((4, 16)), _full_spec((4, 4)),
         _full_spec((4, 4))),
        (16, 16, 2, 2, 4, 4))
    gb1 = _gb(as1, ad1, 4)
    uA, dA = ek_l1(src2d, dst2d, t1a, ad1a.T, gb_splat(gb1, 0, 2), zu8, zd)
    uB, dB = ek_l1(src2d, dst2d, t1b, ad1b.T, gb_splat(gb1, 2, 2), zu8, zd)
    (h1,) = _grid_call(
        _comb1_body, (uA, dA, uB, dB, p['b1'].reshape(1, 16)),
        (_u_spec(8), _d_spec(2), _u_spec(8), _d_spec(2),
         _full_spec((1, 16))), (16,))
    m1s, v1s = _stats(h1, 16)

    t2 = _grid_call(
        _prep2_body,
        (h1, m1s, v1s, p['g1'].reshape(1, 16), p['be1'].reshape(1, 16), m1,
         p['W2'], p['as2'], p['ad2']),
        (_row_spec(16), _full_spec((1, 16)), _full_spec((1, 16)),
         _full_spec((1, 16)), _full_spec((1, 16)), _row_spec(16),
         _full_spec((16, 64)), _full_spec((4, 16)), _full_spec((4, 16))),
        (24, 24, 24, 24, 1, 1, 1, 1, 4, 4))
    t20, t21, t22, t23, a20, a21, a22, a23, as2, ad2 = t2
    gb2 = _gb(as2, ad2, 4)
    acc2 = [ek_l23(src2d, dst2d, tt, aa.T, gb_splat(gb2, hh, 1), zu16, zd)
            for hh, (tt, aa) in enumerate(zip((t20, t21, t22, t23),
                                              (a20, a21, a22, a23)))]
    (h2,) = _grid_call(
        _comb2_body,
        (acc2[0][0], acc2[0][1], acc2[1][0], acc2[1][1],
         acc2[2][0], acc2[2][1], acc2[3][0], acc2[3][1],
         p['b2'].reshape(1, 64)),
        (_u_spec(16), _d_spec(1)) * 4 + (_full_spec((1, 64)),), (64,))
    m2s, v2s = _stats(h2, 64)

    t2d, t3a, t3b, ad3t, as3, ad3 = _grid_call(
        _prep3_body,
        (h2, m2s, v2s, p['g2'].reshape(1, 64), p['be2'].reshape(1, 64), m2,
         p['W3'], p['as3'], p['ad3']),
        (_row_spec(64), _full_spec((1, 64)), _full_spec((1, 64)),
         _full_spec((1, 64)), _full_spec((1, 64)), _row_spec(64),
         _full_spec((64, 32)), _full_spec((1, 32)), _full_spec((1, 32))),
        (64, 24, 24, 1, 1, 1))
    psum, _psq = _stats(t2d, 64)
    gb3 = _gb(as3, ad3, 1)
    u3a, d3a = ek_l23(src2d, dst2d, t3a, ad3t.T, gb_splat(gb3, 0, 1),
                      zu16, zd)
    u3b, d3b = ek_l23(src2d, dst2d, t3b, ad3t.T, gb_splat(gb3, 0, 1),
                      zu16, zd)
    (a3,) = _grid_call(
        _comb3_body, (u3a, d3a, u3b, d3b, p['b3'].reshape(1, 32)),
        (_u_spec(16), _d_spec(1), _u_spec(16), _d_spec(1),
         _full_spec((1, 32))), (32,))
    m3s, v3s = _stats(a3, 32)

    t4, ad4t, as4, ad4 = _grid_call(
        _prep4_body,
        (a3, m3s, v3s, p['g3'].reshape(1, 32), p['be3'].reshape(1, 32),
         p['W4'], p['as4'], p['ad4']),
        (_row_spec(32), _full_spec((1, 32)), _full_spec((1, 32)),
         _full_spec((1, 32)), _full_spec((1, 32)), _full_spec((32, 1)),
         _full_spec((1, 1)), _full_spec((1, 1))),
        (8, 1, 1, 1))
    gb4 = _gb(as4, ad4, 1)
    u4, d4 = ek_l4(src2d, dst2d, t4, ad4t.T, gb_splat(gb4, 0, 1), zu16, zd)
    (actor,) = _grid_call(
        _comb4_body, (u4, d4, p['b4'].reshape(1, 1)),
        (_u_spec(1), _d_spec(1), _full_spec((1, 1))), (1,))

    prob, value, log_prob = pl.pallas_call(
        _final_body,
        out_shape=(jax.ShapeDtypeStruct((1, N), f32),
                   jax.ShapeDtypeStruct((1, 1), f32),
                   jax.ShapeDtypeStruct((1, N), f32)),
    )(actor[:N].reshape(1, N), psum, p['cw1'], p['cb1'].reshape(1, 32),
      p['cw2'], p['cb2'].reshape(1, 16), p['cw3'], p['cb3'].reshape(1, 1))
    return (prob, value, log_prob)
